# initial kernel scaffold (unmeasured)
import jax
import jax.numpy as jnp
from jax import lax
from jax.experimental import pallas as pl
from jax.experimental.pallas import tpu as pltpu

N_DEV = 8
SQ = 512
D = 1024
NH = 8
DH = 128
CHUNK = SQ // N_DEV
SCALE = 0.08838834764831843


def kernel(x, Wq, Wo, Wk, Wv):
    def body(x_ref, wq_ref, wo_ref, wk_ref, wv_ref, out_ref,
             comm_ref, send_sems, recv_sems):
        my = lax.axis_index("i")
        left = lax.rem(my + (N_DEV - 1), N_DEV)
        right = lax.rem(my + 1, N_DEV)

        barrier_sem = pltpu.get_barrier_semaphore()
        for nbr in (left, right):
            pl.semaphore_signal(
                barrier_sem, inc=1,
                device_id=(nbr,), device_id_type=pl.DeviceIdType.MESH,
            )
        pl.semaphore_wait(barrier_sem, 2)

        xm = x_ref[0]
        q = jnp.dot(xm, wq_ref[...], preferred_element_type=jnp.float32)
        k = jnp.dot(xm, wk_ref[...], preferred_element_type=jnp.float32)
        v = jnp.dot(xm, wv_ref[...], preferred_element_type=jnp.float32)

        outs = []
        for h in range(NH):
            qh = q[:, h * DH:(h + 1) * DH]
            kh = k[:, h * DH:(h + 1) * DH]
            vh = v[:, h * DH:(h + 1) * DH]
            s = lax.dot_general(
                qh, kh, (((1,), (1,)), ((), ())),
                preferred_element_type=jnp.float32,
            ) * SCALE
            m = jnp.max(s, axis=1, keepdims=True)
            p = jnp.exp(s - m)
            l = jnp.sum(p, axis=1, keepdims=True)
            outs.append(jnp.dot(p, vh, preferred_element_type=jnp.float32) / l)
        attn = jnp.concatenate(outs, axis=1)
        out_ref[0] = jnp.dot(attn, wo_ref[...],
                             preferred_element_type=jnp.float32)

        for s in range(N_DEV - 1):
            send_idx = lax.rem(my + (N_DEV - s), N_DEV)
            recv_idx = lax.rem(my + (N_DEV - s - 1), N_DEV)
            rdma = pltpu.make_async_remote_copy(
                src_ref=out_ref.at[0, pl.ds(send_idx * CHUNK, CHUNK), :],
                dst_ref=comm_ref.at[s],
                send_sem=send_sems.at[s],
                recv_sem=recv_sems.at[s],
                device_id=(right,),
                device_id_type=pl.DeviceIdType.MESH,
            )
            rdma.start()
            rdma.wait()
            cur = out_ref[0, pl.ds(recv_idx * CHUNK, CHUNK), :]
            out_ref[0, pl.ds(recv_idx * CHUNK, CHUNK), :] = cur + comm_ref[s]

        for s in range(N_DEV - 1):
            send_idx = lax.rem(my + (N_DEV + 1 - s), N_DEV)
            rdma = pltpu.make_async_remote_copy(
                src_ref=out_ref.at[0, pl.ds(send_idx * CHUNK, CHUNK), :],
                dst_ref=out_ref.at[0, pl.ds(send_idx * CHUNK, CHUNK), :],
                send_sem=send_sems.at[N_DEV - 1 + s],
                recv_sem=recv_sems.at[N_DEV - 1 + s],
                device_id=(right,),
                device_id_type=pl.DeviceIdType.MESH,
            )
            rdma.start()
            rdma.wait()

    return pl.pallas_call(
        body,
        out_shape=jax.ShapeDtypeStruct((1, SQ, D), jnp.float32),
        in_specs=[pl.BlockSpec(memory_space=pltpu.VMEM)] * 5,
        out_specs=pl.BlockSpec(memory_space=pltpu.VMEM),
        scratch_shapes=[
            pltpu.VMEM((N_DEV - 1, CHUNK, D), jnp.float32),
            pltpu.SemaphoreType.DMA((2 * (N_DEV - 1),)),
            pltpu.SemaphoreType.DMA((2 * (N_DEV - 1),)),
        ],
        compiler_params=pltpu.CompilerParams(collective_id=0),
    )(x, Wq, Wk, Wv, Wo)


# baseline (device time: 87473 ns/iter reference)
import jax
import jax.numpy as jnp
from jax import lax
from jax.experimental import pallas as pl
from jax.experimental.pallas import tpu as pltpu

N_DEV = 8
SQ = 512
D = 1024
NH = 8
DH = 128
CHUNK = SQ // N_DEV
SCALE = 0.08838834764831843


def kernel(x, Wq, Wo, Wk, Wv):
    def body(x_ref, wq_ref, wk_ref, wv_ref, wo_ref, out_ref,
             comm_ref, send_sems, recv_sems):
        my = lax.axis_index("i")
        left = lax.rem(my + (N_DEV - 1), N_DEV)
        right = lax.rem(my + 1, N_DEV)

        barrier_sem = pltpu.get_barrier_semaphore()
        for nbr in (left, right):
            pl.semaphore_signal(
                barrier_sem, inc=1,
                device_id=(nbr,), device_id_type=pl.DeviceIdType.MESH,
            )
        pl.semaphore_wait(barrier_sem, 2)

        xm = x_ref[0]
        q = jnp.dot(xm, wq_ref[...], preferred_element_type=jnp.float32)
        k = jnp.dot(xm, wk_ref[...], preferred_element_type=jnp.float32)
        v = jnp.dot(xm, wv_ref[...], preferred_element_type=jnp.float32)

        outs = []
        for h in range(NH):
            qh = q[:, h * DH:(h + 1) * DH]
            kh = k[:, h * DH:(h + 1) * DH]
            vh = v[:, h * DH:(h + 1) * DH]
            s = lax.dot_general(
                qh, kh, (((1,), (1,)), ((), ())),
                preferred_element_type=jnp.float32,
            ) * SCALE
            m = jnp.max(s, axis=1, keepdims=True)
            p = jnp.exp(s - m)
            l = jnp.sum(p, axis=1, keepdims=True)
            outs.append(jnp.dot(p, vh, preferred_element_type=jnp.float32) / l)
        attn = jnp.concatenate(outs, axis=1)
        out_ref[0] = jnp.dot(attn, wo_ref[...],
                             preferred_element_type=jnp.float32)

        for s in range(N_DEV - 1):
            send_idx = lax.rem(my + (N_DEV - s), N_DEV)
            recv_idx = lax.rem(my + (N_DEV - s - 1), N_DEV)
            rdma = pltpu.make_async_remote_copy(
                src_ref=out_ref.at[0, pl.ds(send_idx * CHUNK, CHUNK), :],
                dst_ref=comm_ref.at[s],
                send_sem=send_sems.at[s],
                recv_sem=recv_sems.at[s],
                device_id=(right,),
                device_id_type=pl.DeviceIdType.MESH,
            )
            rdma.start()
            rdma.wait()
            cur = out_ref[0, pl.ds(recv_idx * CHUNK, CHUNK), :]
            out_ref[0, pl.ds(recv_idx * CHUNK, CHUNK), :] = cur + comm_ref[s]

        for s in range(N_DEV - 1):
            send_idx = lax.rem(my + (N_DEV + 1 - s), N_DEV)
            rdma = pltpu.make_async_remote_copy(
                src_ref=out_ref.at[0, pl.ds(send_idx * CHUNK, CHUNK), :],
                dst_ref=out_ref.at[0, pl.ds(send_idx * CHUNK, CHUNK), :],
                send_sem=send_sems.at[N_DEV - 1 + s],
                recv_sem=recv_sems.at[N_DEV - 1 + s],
                device_id=(right,),
                device_id_type=pl.DeviceIdType.MESH,
            )
            rdma.start()
            rdma.wait()

    return pl.pallas_call(
        body,
        out_shape=jax.ShapeDtypeStruct((1, SQ, D), jnp.float32),
        in_specs=[pl.BlockSpec(memory_space=pltpu.VMEM)] * 5,
        out_specs=pl.BlockSpec(memory_space=pltpu.VMEM),
        scratch_shapes=[
            pltpu.VMEM((N_DEV - 1, CHUNK, D), jnp.float32),
            pltpu.SemaphoreType.DMA((2 * (N_DEV - 1),)),
            pltpu.SemaphoreType.DMA((2 * (N_DEV - 1),)),
        ],
        compiler_params=pltpu.CompilerParams(collective_id=0),
    )(x, Wq, Wk, Wv, Wo)


# device time: 53029 ns/iter; 1.6495x vs baseline; 1.6495x over previous
import jax
import jax.numpy as jnp
from jax import lax
from jax.experimental import pallas as pl
from jax.experimental.pallas import tpu as pltpu

N_DEV = 8
SQ = 512
D = 1024
NH = 8
DH = 128
SCALE = 0.08838834764831843

_RS_STEPS = ((4, 256), (3, 128), (1, 64))
_AG_STEPS = ((1, 64), (3, 128), (4, 256))


def kernel(x, Wq, Wo, Wk, Wv):
    def body(x_ref, wq_ref, wk_ref, wv_ref, wo_ref, out_ref,
             sbuf, comm0, comm1, comm2, gbuf, send_sems, recv_sems):
        my = lax.axis_index("i")

        barrier_sem = pltpu.get_barrier_semaphore()
        for mask in (1, 3, 4):
            pl.semaphore_signal(
                barrier_sem, inc=1,
                device_id=(jnp.bitwise_xor(my, mask),),
                device_id_type=pl.DeviceIdType.MESH,
            )
        pl.semaphore_wait(barrier_sem, 3)

        xb = x_ref[0].astype(jnp.bfloat16)
        q = jnp.dot(xb, wq_ref[...].astype(jnp.bfloat16),
                    preferred_element_type=jnp.float32)
        k = jnp.dot(xb, wk_ref[...].astype(jnp.bfloat16),
                    preferred_element_type=jnp.float32)
        v = jnp.dot(xb, wv_ref[...].astype(jnp.bfloat16),
                    preferred_element_type=jnp.float32)

        outs = []
        for h in range(NH):
            qh = q[:, h * DH:(h + 1) * DH].astype(jnp.bfloat16)
            kh = k[:, h * DH:(h + 1) * DH].astype(jnp.bfloat16)
            vh = v[:, h * DH:(h + 1) * DH].astype(jnp.bfloat16)
            s = lax.dot_general(
                qh, kh, (((1,), (1,)), ((), ())),
                preferred_element_type=jnp.float32,
            ) * SCALE
            m = jnp.max(s, axis=1, keepdims=True)
            p = jnp.exp(s - m)
            l = jnp.sum(p, axis=1, keepdims=True)
            oh = jnp.dot(p.astype(jnp.bfloat16), vh,
                         preferred_element_type=jnp.float32) / l
            outs.append(oh)
        attn = jnp.concatenate(outs, axis=1).astype(jnp.bfloat16)
        out_ref[0] = jnp.dot(attn, wo_ref[...].astype(jnp.bfloat16),
                             preferred_element_type=jnp.float32)

        comms = (comm0, comm1, comm2)
        lo = jnp.int32(0)
        for step, (mask, half) in enumerate(_RS_STEPS):
            partner = jnp.bitwise_xor(my, mask)
            i_am_high = partner < my
            keep_lo = lo + jnp.where(i_am_high, half, 0).astype(jnp.int32)
            send_lo = lo + jnp.where(i_am_high, 0, half).astype(jnp.int32)
            sbuf[0:half, :] = out_ref[0, pl.ds(send_lo, half), :].astype(
                jnp.bfloat16)
            rdma = pltpu.make_async_remote_copy(
                src_ref=sbuf.at[0:half, :],
                dst_ref=comms[step],
                send_sem=send_sems.at[step],
                recv_sem=recv_sems.at[step],
                device_id=(partner,),
                device_id_type=pl.DeviceIdType.MESH,
            )
            rdma.start()
            rdma.wait()
            cur = out_ref[0, pl.ds(keep_lo, half), :]
            out_ref[0, pl.ds(keep_lo, half), :] = (
                cur + comms[step][...].astype(jnp.float32))
            lo = keep_lo

        gbuf[pl.ds(lo, 64), :] = out_ref[0, pl.ds(lo, 64), :].astype(
            jnp.bfloat16)
        for step, (mask, size) in enumerate(_AG_STEPS):
            partner = jnp.bitwise_xor(my, mask)
            i_am_high = partner < my
            rdma = pltpu.make_async_remote_copy(
                src_ref=gbuf.at[pl.ds(lo, size), :],
                dst_ref=gbuf.at[pl.ds(lo, size), :],
                send_sem=send_sems.at[3 + step],
                recv_sem=recv_sems.at[3 + step],
                device_id=(partner,),
                device_id_type=pl.DeviceIdType.MESH,
            )
            rdma.start()
            rdma.wait()
            lo = lo - jnp.where(i_am_high, size, 0).astype(jnp.int32)
        out_ref[0] = gbuf[...].astype(jnp.float32)

    return pl.pallas_call(
        body,
        out_shape=jax.ShapeDtypeStruct((1, SQ, D), jnp.float32),
        in_specs=[pl.BlockSpec(memory_space=pltpu.VMEM)] * 5,
        out_specs=pl.BlockSpec(memory_space=pltpu.VMEM),
        scratch_shapes=[
            pltpu.VMEM((256, D), jnp.bfloat16),
            pltpu.VMEM((256, D), jnp.bfloat16),
            pltpu.VMEM((128, D), jnp.bfloat16),
            pltpu.VMEM((64, D), jnp.bfloat16),
            pltpu.VMEM((SQ, D), jnp.bfloat16),
            pltpu.SemaphoreType.DMA((6,)),
            pltpu.SemaphoreType.DMA((6,)),
        ],
        compiler_params=pltpu.CompilerParams(collective_id=0),
    )(x, Wq, Wk, Wv, Wo)


# device time: 42033 ns/iter; 2.0811x vs baseline; 1.2616x over previous
import jax
import jax.numpy as jnp
from jax import lax
from jax.experimental import pallas as pl
from jax.experimental.pallas import tpu as pltpu

N_DEV = 8
SQ = 512
D = 1024
NH = 8
DH = 128
SCALE = 0.08838834764831843

_PARTS = ((0, 256), (256, 128), (384, 128))
_RS_MASKS = ((4, 3, 1), (3, 1, 4), (1, 4, 3))
_AG_MASKS = ((1, 3, 4), (4, 1, 3), (3, 4, 1))
_OFFS = ((0, 128, 192), (0, 64, 96), (0, 32, 48))


def kernel(x, Wq, Wo, Wk, Wv):
    def body(x_ref, wq_ref, wk_ref, wv_ref, wo_ref, out_ref,
             acc, comm0, comm1, comm2, send_sems, recv_sems):
        my = lax.axis_index("i")

        def i_am_high(mask):
            if mask == 1:
                bit = jnp.bitwise_and(jnp.bitwise_xor(my, my // 2), 1)
            elif mask == 3:
                bit = jnp.bitwise_and(my // 2, 1)
            else:
                bit = jnp.bitwise_and(my // 4, 1)
            return bit == 1

        barrier_sem = pltpu.get_barrier_semaphore()
        for mask in (1, 3, 4):
            pl.semaphore_signal(
                barrier_sem, inc=1,
                device_id=(jnp.bitwise_xor(my, mask),),
                device_id_type=pl.DeviceIdType.MESH,
            )
        pl.semaphore_wait(barrier_sem, 3)

        xb = x_ref[0].astype(jnp.bfloat16)
        q = jnp.dot(xb, wq_ref[...].astype(jnp.bfloat16),
                    preferred_element_type=jnp.float32) * SCALE
        k = jnp.dot(xb, wk_ref[...].astype(jnp.bfloat16),
                    preferred_element_type=jnp.float32)
        v = jnp.dot(xb, wv_ref[...].astype(jnp.bfloat16),
                    preferred_element_type=jnp.float32)

        outs = []
        for h in range(NH):
            qh = q[:, h * DH:(h + 1) * DH].astype(jnp.bfloat16)
            kh = k[:, h * DH:(h + 1) * DH].astype(jnp.bfloat16)
            vh = v[:, h * DH:(h + 1) * DH].astype(jnp.bfloat16)
            s = lax.dot_general(
                qh, kh, (((1,), (1,)), ((), ())),
                preferred_element_type=jnp.float32,
            )
            p = jnp.exp(s)
            r = 1.0 / jnp.sum(p, axis=1, keepdims=True)
            oh = jnp.dot(p.astype(jnp.bfloat16), vh,
                         preferred_element_type=jnp.float32) * r
            outs.append(oh.astype(jnp.bfloat16))
        attn = jnp.concatenate(outs, axis=1)

        comms = (comm0, comm1, comm2)
        n_parts = len(_PARTS)
        wo_b = wo_ref[...].astype(jnp.bfloat16)

        def start_rs(p, phase, lo_p):
            rows = _PARTS[p][1] >> (phase + 1)
            mask = _RS_MASKS[p][phase]
            off = _OFFS[phase][p]
            high = i_am_high(mask)
            keep_lo = lo_p + jnp.where(high, rows, 0).astype(jnp.int32)
            send_lo = lo_p + jnp.where(high, 0, rows).astype(jnp.int32)
            rdma = pltpu.make_async_remote_copy(
                src_ref=acc.at[pl.ds(send_lo, rows), :],
                dst_ref=comms[phase].at[off:off + rows, :],
                send_sem=send_sems.at[phase * n_parts + p],
                recv_sem=recv_sems.at[phase * n_parts + p],
                device_id=(jnp.bitwise_xor(my, mask),),
                device_id_type=pl.DeviceIdType.MESH,
            )
            rdma.start()
            return rdma, keep_lo

        def add_rs(p, phase, keep_lo):
            rows = _PARTS[p][1] >> (phase + 1)
            off = _OFFS[phase][p]
            cur = acc[pl.ds(keep_lo, rows), :]
            acc[pl.ds(keep_lo, rows), :] = cur + comms[phase][
                off:off + rows, :]

        def start_ag(p, phase, lo_p):
            rows = _PARTS[p][1] >> (3 - phase)
            mask = _AG_MASKS[p][phase]
            rdma = pltpu.make_async_remote_copy(
                src_ref=acc.at[pl.ds(lo_p, rows), :],
                dst_ref=acc.at[pl.ds(lo_p, rows), :],
                send_sem=send_sems.at[(3 + phase) * n_parts + p],
                recv_sem=recv_sems.at[(3 + phase) * n_parts + p],
                device_id=(jnp.bitwise_xor(my, mask),),
                device_id_type=pl.DeviceIdType.MESH,
            )
            rdma.start()
            new_lo = lo_p - jnp.where(
                i_am_high(mask), rows, 0).astype(jnp.int32)
            return rdma, new_lo

        rdmas = [None] * n_parts
        lo = [None] * n_parts
        for p, (base, nr) in enumerate(_PARTS):
            colf = jnp.dot(attn[base:base + nr, :], wo_b,
                           preferred_element_type=jnp.float32)
            acc[base:base + nr, :] = colf.astype(jnp.bfloat16)
            rdmas[p], lo[p] = start_rs(p, 0, jnp.int32(base))

        for phase in (1, 2):
            for p in range(n_parts):
                rdmas[p].wait()
                add_rs(p, phase - 1, lo[p])
                rdmas[p], lo[p] = start_rs(p, phase, lo[p])
        for p in range(n_parts):
            rdmas[p].wait()
            add_rs(p, 2, lo[p])
            rdmas[p], lo[p] = start_ag(p, 0, lo[p])
        for phase in (1, 2):
            for p in range(n_parts):
                rdmas[p].wait()
                rdmas[p], lo[p] = start_ag(p, phase, lo[p])
        for p in range(n_parts):
            rdmas[p].wait()

        out_ref[0] = acc[...].astype(jnp.float32)

    return pl.pallas_call(
        body,
        out_shape=jax.ShapeDtypeStruct((1, SQ, D), jnp.float32),
        in_specs=[pl.BlockSpec(memory_space=pltpu.VMEM)] * 5,
        out_specs=pl.BlockSpec(memory_space=pltpu.VMEM),
        scratch_shapes=[
            pltpu.VMEM((SQ, D), jnp.bfloat16),
            pltpu.VMEM((256, D), jnp.bfloat16),
            pltpu.VMEM((128, D), jnp.bfloat16),
            pltpu.VMEM((64, D), jnp.bfloat16),
            pltpu.SemaphoreType.DMA((18,)),
            pltpu.SemaphoreType.DMA((18,)),
        ],
        compiler_params=pltpu.CompilerParams(collective_id=0),
    )(x, Wq, Wk, Wv, Wo)
